# Initial kernel scaffold; baseline (speedup 1.0000x reference)
import functools

import jax
import jax.numpy as jnp
from jax import lax
from jax.experimental import pallas as pl
from jax.experimental.pallas import tpu as pltpu
from jax.experimental.pallas import tpu_sc as plsc

EMBED = 128
CHUNK = 128
NC, NS = 2, 16
NW = NC * NS


@functools.partial(jax.jit, static_argnames=("n_total",))
def _sc_gather(idx2d, table, n_total):
    b_per_w = n_total // NW
    n_chunks = b_per_w // CHUNK
    mesh = plsc.VectorSubcoreMesh(core_axis_name="c", subcore_axis_name="s")

    @functools.partial(
        pl.kernel,
        mesh=mesh,
        out_type=jax.ShapeDtypeStruct((n_total, EMBED), jnp.float32),
        scratch_types=[
            pltpu.VMEM((n_chunks, CHUNK), jnp.int32),
            pltpu.VMEM((CHUNK, EMBED), jnp.float32),
            pltpu.SemaphoreType.DMA,
        ],
    )
    def k(idx_hbm, table_hbm, out_hbm, idx_v, rows_v, sem):
        wid = lax.axis_index("s") * NC + lax.axis_index("c")
        base = wid * b_per_w
        pltpu.sync_copy(idx_hbm.at[pl.ds(wid * n_chunks, n_chunks)], idx_v)

        def body(j, _):
            pltpu.async_copy(table_hbm.at[idx_v.at[j]], rows_v, sem).wait()
            pltpu.sync_copy(rows_v, out_hbm.at[pl.ds(base + j * CHUNK, CHUNK)])
            return 0

        lax.fori_loop(0, n_chunks, body, 0)

    return k(idx2d, table)


def kernel(inputs, table):
    batch, seq = inputs.shape
    n_total = batch * seq
    idx2d = inputs.reshape(n_total // CHUNK, CHUNK)
    out = _sc_gather(idx2d, table, n_total)
    return out.reshape(batch, seq, EMBED)


# speedup vs baseline: 2.9743x; 2.9743x over previous
import functools

import jax
import jax.numpy as jnp
from jax import lax
from jax.experimental import pallas as pl
from jax.experimental.pallas import tpu as pltpu
from jax.experimental.pallas import tpu_sc as plsc

EMBED = 128
CHUNK = 128
NC, NS = 2, 16
NW = NC * NS


@functools.partial(jax.jit, static_argnames=("n_total",))
def _sc_gather(idx2d, table, n_total):
    b_per_w = n_total // NW
    n_chunks = b_per_w // CHUNK
    mesh = plsc.VectorSubcoreMesh(core_axis_name="c", subcore_axis_name="s")

    @functools.partial(
        pl.kernel,
        mesh=mesh,
        out_type=jax.ShapeDtypeStruct((n_total, EMBED), jnp.float32),
        scratch_types=[
            pltpu.VMEM((b_per_w,), jnp.int32),
            pltpu.VMEM((CHUNK, EMBED), jnp.float32),
            pltpu.SemaphoreType.DMA,
        ],
    )
    def k(idx_hbm, table_hbm, out_hbm, idx_v, rows_v, sem):
        wid = lax.axis_index("s") * NC + lax.axis_index("c")
        base = wid * b_per_w
        pltpu.sync_copy(idx_hbm.at[pl.ds(base, b_per_w)], idx_v)

        def body(j, _):
            pltpu.async_copy(
                table_hbm.at[idx_v.at[pl.ds(j * CHUNK, CHUNK)]], rows_v, sem
            ).wait()
            pltpu.sync_copy(rows_v, out_hbm.at[pl.ds(base + j * CHUNK, CHUNK)])
            return 0

        lax.fori_loop(0, n_chunks, body, 0)

    return k(idx2d, table)


def kernel(inputs, table):
    batch, seq = inputs.shape
    n_total = batch * seq
    idx_flat = inputs.reshape(n_total)
    out = _sc_gather(idx_flat, table, n_total)
    return out.reshape(batch, seq, EMBED)


# speedup vs baseline: 3.3496x; 1.1262x over previous
import functools

import jax
import jax.numpy as jnp
from jax import lax
from jax.experimental import pallas as pl
from jax.experimental.pallas import tpu as pltpu
from jax.experimental.pallas import tpu_sc as plsc

EMBED = 128
CHUNK = 128
NC, NS = 2, 16
NW = NC * NS
NBUF = 5


@functools.partial(jax.jit, static_argnames=("n_total",))
def _sc_gather(idx2d, table, n_total):
    b_per_w = n_total // NW
    n_chunks = b_per_w // CHUNK
    n_outer = n_chunks // NBUF
    mesh = plsc.VectorSubcoreMesh(core_axis_name="c", subcore_axis_name="s")

    @functools.partial(
        pl.kernel,
        mesh=mesh,
        out_type=jax.ShapeDtypeStruct((n_total, EMBED), jnp.float32),
        scratch_types=[
            pltpu.VMEM((b_per_w,), jnp.int32),
            pltpu.VMEM((NBUF, CHUNK, EMBED), jnp.float32),
        ]
        + [pltpu.SemaphoreType.DMA] * (2 * NBUF),
    )
    def k(idx_hbm, table_hbm, out_hbm, idx_v, rows_v, *sems):
        gsem, ssem = sems[:NBUF], sems[NBUF:]
        wid = lax.axis_index("s") * NC + lax.axis_index("c")
        base = wid * b_per_w
        pltpu.sync_copy(idx_hbm.at[pl.ds(base, b_per_w)], idx_v)

        def gather(b, j):
            return pltpu.make_async_copy(
                table_hbm.at[idx_v.at[pl.ds(j * CHUNK, CHUNK)]],
                rows_v.at[b],
                gsem[b],
            )

        def store(b, j):
            return pltpu.make_async_copy(
                rows_v.at[b],
                out_hbm.at[pl.ds(base + j * CHUNK, CHUNK)],
                ssem[b],
            )

        for b in range(NBUF):
            gather(b, b).start()

        def outer(t, _):
            for b in range(NBUF):
                j = t * NBUF + b
                gather(b, j).wait()
                store(b, j).start()
                store(b, j).wait()

                @pl.when(t < n_outer - 1)
                def _():
                    gather(b, j + NBUF).start()

            return 0

        lax.fori_loop(0, n_outer, outer, 0)

    return k(idx2d, table)


def kernel(inputs, table):
    batch, seq = inputs.shape
    n_total = batch * seq
    idx_flat = inputs.reshape(n_total)
    out = _sc_gather(idx_flat, table, n_total)
    return out.reshape(batch, seq, EMBED)


# speedup vs baseline: 5.9652x; 1.7809x over previous
import functools

import jax
import jax.numpy as jnp
from jax import lax
from jax.experimental import pallas as pl
from jax.experimental.pallas import tpu as pltpu
from jax.experimental.pallas import tpu_sc as plsc

EMBED = 128
NC, NS = 2, 16
NW = NC * NS
NBUF = 8


@jax.jit
def _sc_gather(idx, table):
    batch, seq = idx.shape
    rows_per_w = batch // NW
    n_outer = rows_per_w // NBUF
    mesh = plsc.VectorSubcoreMesh(core_axis_name="c", subcore_axis_name="s")

    @functools.partial(
        pl.kernel,
        mesh=mesh,
        out_type=jax.ShapeDtypeStruct((batch, seq, EMBED), jnp.float32),
        scratch_types=[
            pltpu.VMEM((rows_per_w, seq), jnp.int32),
            pltpu.VMEM((NBUF, seq, EMBED), jnp.float32),
        ]
        + [pltpu.SemaphoreType.DMA] * (2 * NBUF),
    )
    def k(idx_hbm, table_hbm, out_hbm, idx_v, rows_v, *sems):
        gsem, ssem = sems[:NBUF], sems[NBUF:]
        wid = lax.axis_index("s") * NC + lax.axis_index("c")
        base = wid * rows_per_w
        pltpu.sync_copy(idx_hbm.at[pl.ds(base, rows_per_w)], idx_v)

        def gather(slot, r):
            return pltpu.make_async_copy(
                table_hbm.at[idx_v.at[r]], rows_v.at[slot], gsem[slot]
            )

        def store(slot, r):
            return pltpu.make_async_copy(
                rows_v.at[slot], out_hbm.at[base + r], ssem[slot]
            )

        for slot in range(NBUF):
            gather(slot, slot).start()

        def outer(t, _):
            for slot in range(NBUF):
                r = t * NBUF + slot
                gather(slot, r).wait()
                store(slot, r).start()
                store(slot, r).wait()

                @pl.when(t < n_outer - 1)
                def _():
                    gather(slot, r + NBUF).start()

            return 0

        lax.fori_loop(0, n_outer, outer, 0)

    return k(idx, table)


def kernel(inputs, table):
    return _sc_gather(inputs, table)


# speedup vs baseline: 10.7801x; 1.8072x over previous
import functools

import jax
import jax.numpy as jnp
from jax import lax
from jax.experimental import pallas as pl
from jax.experimental.pallas import tpu as pltpu
from jax.experimental.pallas import tpu_sc as plsc

EMBED = 128
BLOCK = 128
NC, NS = 2, 16
NW = NC * NS
NBUF = 5


@jax.jit
def _sc_gather(idx_t, table):
    seq, batch = idx_t.shape
    mesh = plsc.VectorSubcoreMesh(core_axis_name="c", subcore_axis_name="s")

    @functools.partial(
        pl.kernel,
        mesh=mesh,
        out_type=jax.ShapeDtypeStruct((seq, batch, EMBED), jnp.float32),
        scratch_types=[
            pltpu.VMEM((seq, BLOCK), jnp.int32),
            pltpu.VMEM((NBUF, BLOCK, EMBED), jnp.float32),
        ]
        + [pltpu.SemaphoreType.DMA] * (2 * NBUF),
    )
    def k(idx_hbm, table_hbm, out_hbm, idx_v, rows_v, *sems):
        gsem, ssem = sems[:NBUF], sems[NBUF:]
        wid = lax.axis_index("s") * NC + lax.axis_index("c")
        col0 = wid * BLOCK
        pltpu.sync_copy(idx_hbm.at[:, pl.ds(col0, BLOCK)], idx_v)

        def gather(slot, s):
            return pltpu.make_async_copy(
                table_hbm.at[idx_v.at[s]], rows_v.at[slot], gsem[slot]
            )

        def store(slot, s):
            return pltpu.make_async_copy(
                rows_v.at[slot],
                out_hbm.at[s].at[pl.ds(col0, BLOCK)],
                ssem[slot],
            )

        for slot in range(NBUF):
            gather(slot, slot).start()

        n_outer = seq // NBUF

        def outer(t, _):
            for slot in range(NBUF):
                s = t * NBUF + slot
                gather(slot, s).wait()
                store(slot, s).start()
                store(slot, s).wait()

                @pl.when(t < n_outer - 1)
                def _():
                    gather(slot, s + NBUF).start()

            return 0

        lax.fori_loop(0, n_outer, outer, 0)

    return k(idx_t, table)


def kernel(inputs, table):
    out = _sc_gather(inputs.T, table)
    return out.transpose(1, 0, 2)
